# baseline (speedup 1.0000x reference)
import functools

import jax
import jax.numpy as jnp
from jax import lax
from jax.experimental import pallas as pl
from jax.experimental.pallas import tpu as pltpu
from jax.experimental.pallas import tpu_sc as plsc

N_AID = 1000000
N_FACTORS = 32
BATCH = 16384

TC_W = 4096
TC_ROWS = TC_W // 4
TC_GRID = (N_AID + TC_W - 1) // TC_W
LIN_ROWS = TC_GRID * TC_ROWS

NUM_CORES = 2
NUM_SUBCORES = 16
NUM_WORKERS = NUM_CORES * NUM_SUBCORES
B_PER_W = BATCH // NUM_WORKERS
ROUND = 256
LANES = 16


def _linearize_tc(tt_ref, out_ref):
    blk = tt_ref[...]
    eye = jnp.eye(N_FACTORS, dtype=jnp.float32)
    parts = [
        lax.dot_general(
            blk[:, q * TC_ROWS:(q + 1) * TC_ROWS], eye,
            (((0,), (0,)), ((), ())),
            preferred_element_type=jnp.float32)
        for q in range(4)
    ]
    out_ref[...] = jnp.concatenate(parts, axis=1)


def _lin_table(table_t):
    return pl.pallas_call(
        _linearize_tc,
        grid=(TC_GRID,),
        in_specs=[pl.BlockSpec((N_FACTORS, TC_W), lambda c: (0, c))],
        out_specs=pl.BlockSpec((TC_ROWS, 128), lambda c: (c, 0)),
        out_shape=jax.ShapeDtypeStruct((LIN_ROWS, 128), jnp.float32),
    )(table_t)


def _mf_kernel(lin_hbm, aid_x_hbm, aid_y_hbm, coef_x_hbm, coef_y_hbm,
               out_hbm,
               idx_x_v, idx_y_v, gx_v, gy_v, cbx_v, cby_v,
               rows_x_v, rows_y_v, cx_v, cy_v, out_v, sem_x, sem_y):
    wid = lax.axis_index("s") * NUM_CORES + lax.axis_index("c")
    base = wid * B_PER_W
    chunk = pl.ds(base, B_PER_W)

    pltpu.sync_copy(aid_x_hbm.at[chunk], idx_x_v)
    pltpu.sync_copy(aid_y_hbm.at[chunk], idx_y_v)
    pltpu.sync_copy(coef_x_hbm.at[chunk], cx_v)
    pltpu.sync_copy(coef_y_hbm.at[chunk], cy_v)

    for c in range(B_PER_W // LANES):
        sl = pl.ds(c * LANES, LANES)
        ax = idx_x_v[sl]
        ay = idx_y_v[sl]
        gx_v[sl] = jnp.bitwise_or(
            lax.shift_left(lax.shift_right_logical(ax, 12), 10),
            jnp.bitwise_and(ax, 1023))
        gy_v[sl] = jnp.bitwise_or(
            lax.shift_left(lax.shift_right_logical(ay, 12), 10),
            jnp.bitwise_and(ay, 1023))
        cbx_v[sl] = lax.shift_left(
            jnp.bitwise_and(lax.shift_right_logical(ax, 10), 3), 5)
        cby_v[sl] = lax.shift_left(
            jnp.bitwise_and(lax.shift_right_logical(ay, 10), 3), 5)

    lane_iota = lax.iota(jnp.int32, LANES)

    for r in range(B_PER_W // ROUND):
        rsl = pl.ds(r * ROUND, ROUND)
        cpx = pltpu.async_copy(lin_hbm.at[gx_v.at[rsl]], rows_x_v, sem_x)
        cpy = pltpu.async_copy(lin_hbm.at[gy_v.at[rsl]], rows_y_v, sem_y)
        cpx.wait()
        cpy.wait()

        def round_body(c, _):
            lsl = pl.ds(r * ROUND + c * LANES, LANES)
            rows = c * LANES + lane_iota
            colx = cbx_v[lsl]
            coly = cby_v[lsl]
            acc = jnp.zeros((LANES,), jnp.float32)
            for j in range(N_FACTORS):
                xv = plsc.load_gather(rows_x_v, [rows, colx + j])
                yv = plsc.load_gather(rows_y_v, [rows, coly + j])
                acc = acc + xv * yv
            out_v[lsl] = acc * cx_v[lsl] * cy_v[lsl]
            return _

        lax.fori_loop(0, ROUND // LANES, round_body, 0)

    pltpu.sync_copy(out_v, out_hbm.at[chunk])


@jax.jit
def kernel(aid_x, aid_y, coef_x, coef_y, aid_embeddings):
    lin = _lin_table(aid_embeddings.T)
    mesh = plsc.VectorSubcoreMesh(
        core_axis_name="c", subcore_axis_name="s",
        num_cores=NUM_CORES, num_subcores=NUM_SUBCORES)
    run = functools.partial(
        pl.kernel,
        out_type=jax.ShapeDtypeStruct((BATCH,), jnp.float32),
        mesh=mesh,
        compiler_params=pltpu.CompilerParams(needs_layout_passes=False),
        scratch_types=[
            pltpu.VMEM((B_PER_W,), jnp.int32),
            pltpu.VMEM((B_PER_W,), jnp.int32),
            pltpu.VMEM((B_PER_W,), jnp.int32),
            pltpu.VMEM((B_PER_W,), jnp.int32),
            pltpu.VMEM((B_PER_W,), jnp.int32),
            pltpu.VMEM((B_PER_W,), jnp.int32),
            pltpu.VMEM((ROUND, 128), jnp.float32),
            pltpu.VMEM((ROUND, 128), jnp.float32),
            pltpu.VMEM((B_PER_W,), jnp.float32),
            pltpu.VMEM((B_PER_W,), jnp.float32),
            pltpu.VMEM((B_PER_W,), jnp.float32),
            pltpu.SemaphoreType.DMA,
            pltpu.SemaphoreType.DMA,
        ],
    )(_mf_kernel)
    return run(lin, aid_x.astype(jnp.int32), aid_y.astype(jnp.int32),
               coef_x, coef_y)


# speedup vs baseline: 1.3788x; 1.3788x over previous
import functools

import jax
import jax.numpy as jnp
from jax import lax
from jax.experimental import pallas as pl
from jax.experimental.pallas import tpu as pltpu
from jax.experimental.pallas import tpu_sc as plsc

N_AID = 1000000
N_FACTORS = 32
BATCH = 16384

TC_W = 4096
TC_ROWS = TC_W // 4
TC_GRID = (N_AID + TC_W - 1) // TC_W
LIN_ROWS = TC_GRID * TC_ROWS

NUM_CORES = 2
NUM_SUBCORES = 16
NUM_WORKERS = NUM_CORES * NUM_SUBCORES
B_PER_W = BATCH // NUM_WORKERS
ROUND = 256
LANES = 16


def _linearize_tc(tt_ref, out_ref):
    blk = tt_ref[...]
    stacked = jnp.concatenate(
        [blk[:, q * TC_ROWS:(q + 1) * TC_ROWS] for q in range(4)], axis=0)
    eye = jnp.eye(128, dtype=jnp.float32)
    out_ref[...] = lax.dot_general(
        stacked, eye, (((0,), (0,)), ((), ())),
        preferred_element_type=jnp.float32)


def _lin_table(table_t):
    return pl.pallas_call(
        _linearize_tc,
        grid=(TC_GRID,),
        in_specs=[pl.BlockSpec((N_FACTORS, TC_W), lambda c: (0, c))],
        out_specs=pl.BlockSpec((TC_ROWS, 128), lambda c: (c, 0)),
        out_shape=jax.ShapeDtypeStruct((LIN_ROWS, 128), jnp.float32),
    )(table_t)


def _mf_kernel(lin_hbm, aid_x_hbm, aid_y_hbm, coef_x_hbm, coef_y_hbm,
               out_hbm,
               idx_x_v, idx_y_v, gx_v, gy_v, cbx_v, cby_v,
               rows_x_v, rows_y_v, cx_v, cy_v, out_v, sem_x, sem_y):
    wid = lax.axis_index("s") * NUM_CORES + lax.axis_index("c")
    base = wid * B_PER_W
    chunk = pl.ds(base, B_PER_W)

    pltpu.sync_copy(aid_x_hbm.at[chunk], idx_x_v)
    pltpu.sync_copy(aid_y_hbm.at[chunk], idx_y_v)
    pltpu.sync_copy(coef_x_hbm.at[chunk], cx_v)
    pltpu.sync_copy(coef_y_hbm.at[chunk], cy_v)

    for c in range(B_PER_W // LANES):
        sl = pl.ds(c * LANES, LANES)
        ax = idx_x_v[sl]
        ay = idx_y_v[sl]
        gx_v[sl] = jnp.bitwise_or(
            lax.shift_left(lax.shift_right_logical(ax, 12), 10),
            jnp.bitwise_and(ax, 1023))
        gy_v[sl] = jnp.bitwise_or(
            lax.shift_left(lax.shift_right_logical(ay, 12), 10),
            jnp.bitwise_and(ay, 1023))
        cbx_v[sl] = lax.shift_left(
            jnp.bitwise_and(lax.shift_right_logical(ax, 10), 3), 5)
        cby_v[sl] = lax.shift_left(
            jnp.bitwise_and(lax.shift_right_logical(ay, 10), 3), 5)

    lane_iota = lax.iota(jnp.int32, LANES)

    for r in range(B_PER_W // ROUND):
        rsl = pl.ds(r * ROUND, ROUND)
        cpx = pltpu.async_copy(lin_hbm.at[gx_v.at[rsl]], rows_x_v, sem_x)
        cpy = pltpu.async_copy(lin_hbm.at[gy_v.at[rsl]], rows_y_v, sem_y)
        cpx.wait()
        cpy.wait()

        def round_body(c, _):
            lsl = pl.ds(r * ROUND + c * LANES, LANES)
            rows = c * LANES + lane_iota
            colx = cbx_v[lsl]
            coly = cby_v[lsl]
            acc = jnp.zeros((LANES,), jnp.float32)
            for j in range(N_FACTORS):
                xv = plsc.load_gather(rows_x_v, [rows, colx + j])
                yv = plsc.load_gather(rows_y_v, [rows, coly + j])
                acc = acc + xv * yv
            out_v[lsl] = acc * cx_v[lsl] * cy_v[lsl]
            return _

        lax.fori_loop(0, ROUND // LANES, round_body, 0)

    pltpu.sync_copy(out_v, out_hbm.at[chunk])


@jax.jit
def kernel(aid_x, aid_y, coef_x, coef_y, aid_embeddings):
    lin = _lin_table(aid_embeddings.T)
    mesh = plsc.VectorSubcoreMesh(
        core_axis_name="c", subcore_axis_name="s",
        num_cores=NUM_CORES, num_subcores=NUM_SUBCORES)
    run = functools.partial(
        pl.kernel,
        out_type=jax.ShapeDtypeStruct((BATCH,), jnp.float32),
        mesh=mesh,
        compiler_params=pltpu.CompilerParams(needs_layout_passes=False),
        scratch_types=[
            pltpu.VMEM((B_PER_W,), jnp.int32),
            pltpu.VMEM((B_PER_W,), jnp.int32),
            pltpu.VMEM((B_PER_W,), jnp.int32),
            pltpu.VMEM((B_PER_W,), jnp.int32),
            pltpu.VMEM((B_PER_W,), jnp.int32),
            pltpu.VMEM((B_PER_W,), jnp.int32),
            pltpu.VMEM((ROUND, 128), jnp.float32),
            pltpu.VMEM((ROUND, 128), jnp.float32),
            pltpu.VMEM((B_PER_W,), jnp.float32),
            pltpu.VMEM((B_PER_W,), jnp.float32),
            pltpu.VMEM((B_PER_W,), jnp.float32),
            pltpu.SemaphoreType.DMA,
            pltpu.SemaphoreType.DMA,
        ],
    )(_mf_kernel)
    return run(lin, aid_x.astype(jnp.int32), aid_y.astype(jnp.int32),
               coef_x, coef_y)


# speedup vs baseline: 2.3908x; 1.7340x over previous
import functools

import jax
import jax.numpy as jnp
from jax import lax
from jax.experimental import pallas as pl
from jax.experimental.pallas import tpu as pltpu
from jax.experimental.pallas import tpu_sc as plsc

N_AID = 1000000
N_FACTORS = 32
BATCH = 16384

TC_W = 16384
TC_ROWS = TC_W // 4
TC_GRID = (N_AID + TC_W - 1) // TC_W
LIN_ROWS = TC_GRID * TC_ROWS
W_SHIFT = TC_W.bit_length() - 1
R_SHIFT = TC_ROWS.bit_length() - 1
R_MASK = TC_ROWS - 1

NUM_CORES = 2
NUM_SUBCORES = 16
NUM_WORKERS = NUM_CORES * NUM_SUBCORES
B_PER_W = BATCH // NUM_WORKERS
ROUND = 256
LANES = 16


def _linearize_tc(tt_ref, out_ref):
    blk = tt_ref[...]
    stacked = jnp.concatenate(
        [blk[:, q * TC_ROWS:(q + 1) * TC_ROWS] for q in range(4)], axis=0)
    eye = jnp.eye(128, dtype=jnp.float32)
    out_ref[...] = lax.dot_general(
        stacked, eye, (((0,), (0,)), ((), ())),
        preferred_element_type=jnp.float32)


def _lin_table(table_t):
    return pl.pallas_call(
        _linearize_tc,
        grid=(TC_GRID,),
        in_specs=[pl.BlockSpec((N_FACTORS, TC_W), lambda c: (0, c))],
        out_specs=pl.BlockSpec((TC_ROWS, 128), lambda c: (c, 0)),
        out_shape=jax.ShapeDtypeStruct((LIN_ROWS, 128), jnp.float32),
    )(table_t)


def _mf_kernel(lin_hbm, aid_x_hbm, aid_y_hbm, coef_x_hbm, coef_y_hbm,
               out_hbm,
               idx_x_v, idx_y_v, gx_v, gy_v, cbx_v, cby_v,
               rows_x_v, rows_y_v, cx_v, cy_v, out_v, sem_x, sem_y):
    wid = lax.axis_index("s") * NUM_CORES + lax.axis_index("c")
    base = wid * B_PER_W
    chunk = pl.ds(base, B_PER_W)

    pltpu.sync_copy(aid_x_hbm.at[chunk], idx_x_v)
    pltpu.sync_copy(aid_y_hbm.at[chunk], idx_y_v)
    pltpu.sync_copy(coef_x_hbm.at[chunk], cx_v)
    pltpu.sync_copy(coef_y_hbm.at[chunk], cy_v)

    for c in range(B_PER_W // LANES):
        sl = pl.ds(c * LANES, LANES)
        ax = idx_x_v[sl]
        ay = idx_y_v[sl]
        gx_v[sl] = jnp.bitwise_or(
            lax.shift_left(lax.shift_right_logical(ax, W_SHIFT), R_SHIFT),
            jnp.bitwise_and(ax, R_MASK))
        gy_v[sl] = jnp.bitwise_or(
            lax.shift_left(lax.shift_right_logical(ay, W_SHIFT), R_SHIFT),
            jnp.bitwise_and(ay, R_MASK))
        cbx_v[sl] = lax.shift_left(
            jnp.bitwise_and(lax.shift_right_logical(ax, R_SHIFT), 3), 5)
        cby_v[sl] = lax.shift_left(
            jnp.bitwise_and(lax.shift_right_logical(ay, R_SHIFT), 3), 5)

    lane_iota = lax.iota(jnp.int32, LANES)

    for r in range(B_PER_W // ROUND):
        rsl = pl.ds(r * ROUND, ROUND)
        cpx = pltpu.async_copy(lin_hbm.at[gx_v.at[rsl]], rows_x_v, sem_x)
        cpy = pltpu.async_copy(lin_hbm.at[gy_v.at[rsl]], rows_y_v, sem_y)
        cpx.wait()
        cpy.wait()

        def round_body(c, _):
            lsl = pl.ds(r * ROUND + c * LANES, LANES)
            rows = c * LANES + lane_iota
            colx = cbx_v[lsl]
            coly = cby_v[lsl]
            acc = jnp.zeros((LANES,), jnp.float32)
            for j in range(N_FACTORS):
                xv = plsc.load_gather(rows_x_v, [rows, colx + j])
                yv = plsc.load_gather(rows_y_v, [rows, coly + j])
                acc = acc + xv * yv
            out_v[lsl] = acc * cx_v[lsl] * cy_v[lsl]
            return _

        lax.fori_loop(0, ROUND // LANES, round_body, 0)

    pltpu.sync_copy(out_v, out_hbm.at[chunk])


@jax.jit
def kernel(aid_x, aid_y, coef_x, coef_y, aid_embeddings):
    lin = _lin_table(aid_embeddings.T)
    mesh = plsc.VectorSubcoreMesh(
        core_axis_name="c", subcore_axis_name="s",
        num_cores=NUM_CORES, num_subcores=NUM_SUBCORES)
    run = functools.partial(
        pl.kernel,
        out_type=jax.ShapeDtypeStruct((BATCH,), jnp.float32),
        mesh=mesh,
        compiler_params=pltpu.CompilerParams(needs_layout_passes=False),
        scratch_types=[
            pltpu.VMEM((B_PER_W,), jnp.int32),
            pltpu.VMEM((B_PER_W,), jnp.int32),
            pltpu.VMEM((B_PER_W,), jnp.int32),
            pltpu.VMEM((B_PER_W,), jnp.int32),
            pltpu.VMEM((B_PER_W,), jnp.int32),
            pltpu.VMEM((B_PER_W,), jnp.int32),
            pltpu.VMEM((ROUND, 128), jnp.float32),
            pltpu.VMEM((ROUND, 128), jnp.float32),
            pltpu.VMEM((B_PER_W,), jnp.float32),
            pltpu.VMEM((B_PER_W,), jnp.float32),
            pltpu.VMEM((B_PER_W,), jnp.float32),
            pltpu.SemaphoreType.DMA,
            pltpu.SemaphoreType.DMA,
        ],
    )(_mf_kernel)
    return run(lin, aid_x.astype(jnp.int32), aid_y.astype(jnp.int32),
               coef_x, coef_y)


# speedup vs baseline: 2.6670x; 1.1156x over previous
import functools

import jax
import jax.numpy as jnp
from jax import lax
from jax.experimental import pallas as pl
from jax.experimental.pallas import tpu as pltpu
from jax.experimental.pallas import tpu_sc as plsc

N_AID = 1000000
N_FACTORS = 32
BATCH = 16384

TC_W = 32768
TC_ROWS = TC_W // 4
TC_GRID = (N_AID + TC_W - 1) // TC_W
LIN_ROWS = TC_GRID * TC_ROWS
W_SHIFT = TC_W.bit_length() - 1
R_SHIFT = TC_ROWS.bit_length() - 1
R_MASK = TC_ROWS - 1

NUM_CORES = 2
NUM_SUBCORES = 16
NUM_WORKERS = NUM_CORES * NUM_SUBCORES
B_PER_W = BATCH // NUM_WORKERS
ROUND = 256
LANES = 16


def _linearize_tc(tt_ref, out_ref):
    blk = tt_ref[...]
    stacked = jnp.concatenate(
        [blk[:, q * TC_ROWS:(q + 1) * TC_ROWS] for q in range(4)], axis=0)
    eye = jnp.eye(128, dtype=jnp.float32)
    out_ref[...] = lax.dot_general(
        stacked, eye, (((0,), (0,)), ((), ())),
        preferred_element_type=jnp.float32)


def _lin_table(table_t):
    return pl.pallas_call(
        _linearize_tc,
        grid=(TC_GRID,),
        in_specs=[pl.BlockSpec((N_FACTORS, TC_W), lambda c: (0, c))],
        out_specs=pl.BlockSpec((TC_ROWS, 128), lambda c: (c, 0)),
        out_shape=jax.ShapeDtypeStruct((LIN_ROWS, 128), jnp.float32),
    )(table_t)


def _mf_kernel(lin_hbm, aid_x_hbm, aid_y_hbm, coef_x_hbm, coef_y_hbm,
               out_hbm,
               idx_x_v, idx_y_v, gx_v, gy_v, cbx_v, cby_v,
               rows_x_v, rows_y_v, cx_v, cy_v, out_v, sem_x, sem_y):
    wid = lax.axis_index("s") * NUM_CORES + lax.axis_index("c")
    base = wid * B_PER_W
    chunk = pl.ds(base, B_PER_W)

    pltpu.sync_copy(aid_x_hbm.at[chunk], idx_x_v)
    pltpu.sync_copy(aid_y_hbm.at[chunk], idx_y_v)
    pltpu.sync_copy(coef_x_hbm.at[chunk], cx_v)
    pltpu.sync_copy(coef_y_hbm.at[chunk], cy_v)

    for c in range(B_PER_W // LANES):
        sl = pl.ds(c * LANES, LANES)
        ax = idx_x_v[sl]
        ay = idx_y_v[sl]
        gx_v[sl] = jnp.bitwise_or(
            lax.shift_left(lax.shift_right_logical(ax, W_SHIFT), R_SHIFT),
            jnp.bitwise_and(ax, R_MASK))
        gy_v[sl] = jnp.bitwise_or(
            lax.shift_left(lax.shift_right_logical(ay, W_SHIFT), R_SHIFT),
            jnp.bitwise_and(ay, R_MASK))
        cbx_v[sl] = lax.shift_left(
            jnp.bitwise_and(lax.shift_right_logical(ax, R_SHIFT), 3), 5)
        cby_v[sl] = lax.shift_left(
            jnp.bitwise_and(lax.shift_right_logical(ay, R_SHIFT), 3), 5)

    lane_iota = lax.iota(jnp.int32, LANES)

    for r in range(B_PER_W // ROUND):
        rsl = pl.ds(r * ROUND, ROUND)
        cpx = pltpu.async_copy(lin_hbm.at[gx_v.at[rsl]], rows_x_v, sem_x)
        cpy = pltpu.async_copy(lin_hbm.at[gy_v.at[rsl]], rows_y_v, sem_y)
        cpx.wait()
        cpy.wait()

        def round_body(c, _):
            lsl = pl.ds(r * ROUND + c * LANES, LANES)
            rows = c * LANES + lane_iota
            colx = cbx_v[lsl]
            coly = cby_v[lsl]
            acc = jnp.zeros((LANES,), jnp.float32)
            for j in range(N_FACTORS):
                xv = plsc.load_gather(rows_x_v, [rows, colx + j])
                yv = plsc.load_gather(rows_y_v, [rows, coly + j])
                acc = acc + xv * yv
            out_v[lsl] = acc * cx_v[lsl] * cy_v[lsl]
            return _

        lax.fori_loop(0, ROUND // LANES, round_body, 0)

    pltpu.sync_copy(out_v, out_hbm.at[chunk])


@jax.jit
def kernel(aid_x, aid_y, coef_x, coef_y, aid_embeddings):
    lin = _lin_table(aid_embeddings.T)
    mesh = plsc.VectorSubcoreMesh(
        core_axis_name="c", subcore_axis_name="s",
        num_cores=NUM_CORES, num_subcores=NUM_SUBCORES)
    run = functools.partial(
        pl.kernel,
        out_type=jax.ShapeDtypeStruct((BATCH,), jnp.float32),
        mesh=mesh,
        compiler_params=pltpu.CompilerParams(needs_layout_passes=False),
        scratch_types=[
            pltpu.VMEM((B_PER_W,), jnp.int32),
            pltpu.VMEM((B_PER_W,), jnp.int32),
            pltpu.VMEM((B_PER_W,), jnp.int32),
            pltpu.VMEM((B_PER_W,), jnp.int32),
            pltpu.VMEM((B_PER_W,), jnp.int32),
            pltpu.VMEM((B_PER_W,), jnp.int32),
            pltpu.VMEM((ROUND, 128), jnp.float32),
            pltpu.VMEM((ROUND, 128), jnp.float32),
            pltpu.VMEM((B_PER_W,), jnp.float32),
            pltpu.VMEM((B_PER_W,), jnp.float32),
            pltpu.VMEM((B_PER_W,), jnp.float32),
            pltpu.SemaphoreType.DMA,
            pltpu.SemaphoreType.DMA,
        ],
    )(_mf_kernel)
    return run(lin, aid_x.astype(jnp.int32), aid_y.astype(jnp.int32),
               coef_x, coef_y)


# speedup vs baseline: 2.8285x; 1.0605x over previous
import functools

import jax
import jax.numpy as jnp
from jax import lax
from jax.experimental import pallas as pl
from jax.experimental.pallas import tpu as pltpu
from jax.experimental.pallas import tpu_sc as plsc

N_AID = 1000000
N_FACTORS = 32
BATCH = 16384

TC_W = 65536
TC_ROWS = TC_W // 4
TC_GRID = (N_AID + TC_W - 1) // TC_W
LIN_ROWS = TC_GRID * TC_ROWS
W_SHIFT = TC_W.bit_length() - 1
R_SHIFT = TC_ROWS.bit_length() - 1
R_MASK = TC_ROWS - 1

NUM_CORES = 2
NUM_SUBCORES = 16
NUM_WORKERS = NUM_CORES * NUM_SUBCORES
B_PER_W = BATCH // NUM_WORKERS
LANES = 16
BLOCKS = B_PER_W // LANES


def _linearize_tc(tt_ref, out_ref):
    blk = tt_ref[...]
    stacked = jnp.concatenate(
        [blk[:, q * TC_ROWS:(q + 1) * TC_ROWS] for q in range(4)], axis=0)
    eye = jnp.eye(128, dtype=jnp.float32)
    out_ref[...] = lax.dot_general(
        stacked, eye, (((0,), (0,)), ((), ())),
        preferred_element_type=jnp.float32)


def _lin_table(table_t):
    return pl.pallas_call(
        _linearize_tc,
        grid=(TC_GRID,),
        in_specs=[pl.BlockSpec((N_FACTORS, TC_W), lambda c: (0, c))],
        out_specs=pl.BlockSpec((TC_ROWS, 128), lambda c: (c, 0)),
        out_shape=jax.ShapeDtypeStruct((LIN_ROWS, 128), jnp.float32),
    )(table_t)


def _mf_kernel(lin_hbm, aid_x_hbm, aid_y_hbm, coef_x_hbm, coef_y_hbm,
               out_hbm,
               idx_x_v, idx_y_v, rows_x_v, rows_y_v, cx_v, cy_v, out_v,
               sem_x, sem_y):
    wid = lax.axis_index("s") * NUM_CORES + lax.axis_index("c")
    base = wid * B_PER_W
    chunk = pl.ds(base, B_PER_W)

    pltpu.sync_copy(aid_x_hbm.at[chunk], idx_x_v)
    pltpu.sync_copy(aid_y_hbm.at[chunk], idx_y_v)

    for c in range(BLOCKS):
        sl = pl.ds(c * LANES, LANES)
        ax = idx_x_v[sl]
        ay = idx_y_v[sl]
        lx = jnp.bitwise_or(
            lax.shift_left(lax.shift_right_logical(ax, W_SHIFT), R_SHIFT),
            jnp.bitwise_and(ax, R_MASK))
        ly = jnp.bitwise_or(
            lax.shift_left(lax.shift_right_logical(ay, W_SHIFT), R_SHIFT),
            jnp.bitwise_and(ay, R_MASK))
        idx_x_v[sl] = jnp.bitwise_or(
            lax.shift_left(lx, 2),
            jnp.bitwise_and(lax.shift_right_logical(ax, R_SHIFT), 3))
        idx_y_v[sl] = jnp.bitwise_or(
            lax.shift_left(ly, 2),
            jnp.bitwise_and(lax.shift_right_logical(ay, R_SHIFT), 3))

    cpx = pltpu.async_copy(lin_hbm.at[idx_x_v], rows_x_v, sem_x)
    cpy = pltpu.async_copy(lin_hbm.at[idx_y_v], rows_y_v, sem_y)
    pltpu.sync_copy(coef_x_hbm.at[chunk], cx_v)
    pltpu.sync_copy(coef_y_hbm.at[chunk], cy_v)
    cpx.wait()
    cpy.wait()

    lane_iota = lax.iota(jnp.int32, LANES)

    def block_body(b, _):
        rows = b * LANES + lane_iota
        sl = pl.ds(b * LANES, LANES)
        acc = jnp.zeros((LANES,), jnp.float32)
        for j in range(N_FACTORS):
            col = jnp.full((LANES,), j, jnp.int32)
            xv = plsc.load_gather(rows_x_v, [rows, col])
            yv = plsc.load_gather(rows_y_v, [rows, col])
            acc = acc + xv * yv
        out_v[sl] = acc * cx_v[sl] * cy_v[sl]
        return _

    lax.fori_loop(0, BLOCKS, block_body, 0)

    pltpu.sync_copy(out_v, out_hbm.at[chunk])


@jax.jit
def kernel(aid_x, aid_y, coef_x, coef_y, aid_embeddings):
    lin = _lin_table(aid_embeddings.T).reshape(4 * LIN_ROWS, N_FACTORS)
    mesh = plsc.VectorSubcoreMesh(
        core_axis_name="c", subcore_axis_name="s",
        num_cores=NUM_CORES, num_subcores=NUM_SUBCORES)
    run = functools.partial(
        pl.kernel,
        out_type=jax.ShapeDtypeStruct((BATCH,), jnp.float32),
        mesh=mesh,
        compiler_params=pltpu.CompilerParams(
            needs_layout_passes=False, use_tc_tiling_on_sc=False),
        scratch_types=[
            pltpu.VMEM((B_PER_W,), jnp.int32),
            pltpu.VMEM((B_PER_W,), jnp.int32),
            pltpu.VMEM((B_PER_W, N_FACTORS), jnp.float32),
            pltpu.VMEM((B_PER_W, N_FACTORS), jnp.float32),
            pltpu.VMEM((B_PER_W,), jnp.float32),
            pltpu.VMEM((B_PER_W,), jnp.float32),
            pltpu.VMEM((B_PER_W,), jnp.float32),
            pltpu.SemaphoreType.DMA,
            pltpu.SemaphoreType.DMA,
        ],
    )(_mf_kernel)
    return run(lin, aid_x.astype(jnp.int32), aid_y.astype(jnp.int32),
               coef_x, coef_y)
